# Initial kernel scaffold; baseline (speedup 1.0000x reference)
import jax
import jax.numpy as jnp
from jax.experimental import pallas as pl


def kernel(x, edge_index, batch, W1, a_s1, a_d1, b1, W2, a_s2, a_d2, b2, linW, linb):
    raise NotImplementedError("write your pallas kernel here")


# speedup vs baseline: 21.0321x; 21.0321x over previous
import functools

import jax
import jax.numpy as jnp
from jax import lax
from jax.experimental import pallas as pl
from jax.experimental.pallas import tpu as pltpu
from jax.experimental.pallas import tpu_sc as plsc

N = 10000
E = 320000
D = 128
H = 128
G = 16
OUT = 64

NP = 10016
W_COLS = 144
NC = 2
NS = 16
NW = NC * NS
EPT = E // NW
CHUNK = 128
NCH = -(-EPT // CHUNK)
EPT_PAD = NCH * CHUNK
ROWS_PT = NP // NS
NEG = -1e30


def _prep_body(x_ref, w_ref, as_ref, ad_ref, xpw_ref, asrc_ref, adst_ref,
               init_ref):
    i = pl.program_id(0)
    br = x_ref.shape[0]
    xp = jnp.dot(x_ref[...], w_ref[...], preferred_element_type=jnp.float32)
    ones = jnp.ones((br, 1), jnp.float32)
    zpad = jnp.zeros((br, W_COLS - H - 1), jnp.float32)
    xpw_ref[...] = jnp.concatenate([xp, ones, zpad], axis=1)

    rows = i * br + lax.broadcasted_iota(jnp.int32, (br,), 0)
    valid = rows < N
    asrc = jnp.sum(xp * as_ref[...], axis=1)
    adst = jnp.sum(xp * ad_ref[...], axis=1)
    asrc = jnp.where(valid, asrc, NEG)
    adst = jnp.where(valid, adst, NEG)
    asrc_ref[...] = asrc[:, None]
    adst_ref[...] = adst[:, None]

    a = asrc + adst
    a = jnp.where(a >= 0, a, 0.2 * a)
    ex_self = jnp.where(valid, jnp.exp(a), 1.0)
    init0 = jnp.concatenate([xp * ex_self[:, None], ex_self[:, None], zpad],
                            axis=1)
    init_ref[...] = jnp.stack([init0, jnp.zeros_like(init0)], axis=0)


def _prep(x_p, W, a_s, a_d):
    br = NP // 4
    return pl.pallas_call(
        _prep_body,
        grid=(4,),
        in_specs=[
            pl.BlockSpec((br, D), lambda i: (i, 0)),
            pl.BlockSpec((D, H), lambda i: (0, 0)),
            pl.BlockSpec((1, H), lambda i: (0, 0)),
            pl.BlockSpec((1, H), lambda i: (0, 0)),
        ],
        out_specs=[
            pl.BlockSpec((br, W_COLS), lambda i: (i, 0)),
            pl.BlockSpec((br, 1), lambda i: (i, 0)),
            pl.BlockSpec((br, 1), lambda i: (i, 0)),
            pl.BlockSpec((2, br, W_COLS), lambda i: (0, i, 0)),
        ],
        out_shape=[
            jax.ShapeDtypeStruct((NP, W_COLS), jnp.float32),
            jax.ShapeDtypeStruct((NP, 1), jnp.float32),
            jax.ShapeDtypeStruct((NP, 1), jnp.float32),
            jax.ShapeDtypeStruct((2, NP, W_COLS), jnp.float32),
        ],
    )(x_p, W, a_s.reshape(1, H), a_d.reshape(1, H))


def _edge_body(xpw_hbm, asrc_hbm, adst_hbm, src_hbm, dst_hbm, init_hbm,
               out_hbm, asrc_v, adst_v, idx_v, ex_v, rows_v, acc_sh, sem):
    c = lax.axis_index("c")
    s = lax.axis_index("s")
    wid = c * NS + s

    r0 = s * ROWS_PT
    pltpu.sync_copy(init_hbm.at[c, pl.ds(r0, ROWS_PT)],
                    acc_sh.at[pl.ds(r0, ROWS_PT)])
    pltpu.sync_copy(asrc_hbm, asrc_v)
    pltpu.sync_copy(adst_hbm, adst_v)
    plsc.subcore_barrier()

    tile_base = wid * EPT_PAD

    def chunk(ci, _):
        base = tile_base + ci * CHUNK
        pltpu.sync_copy(src_hbm.at[pl.ds(base, CHUNK)], idx_v.at[0])
        pltpu.sync_copy(dst_hbm.at[pl.ds(base, CHUNK)], idx_v.at[1])
        for i in range(CHUNK // 16):
            sv = idx_v[0, pl.ds(i * 16, 16)]
            dv = idx_v[1, pl.ds(i * 16, 16)]
            a = plsc.load_gather(asrc_v, [sv]) + plsc.load_gather(adst_v, [dv])
            a = jnp.where(a >= 0, a, jnp.float32(0.2) * a)
            ex_v[pl.ds(i * 16, 16)] = jnp.exp(a)
        pltpu.async_copy(xpw_hbm.at[idx_v.at[0]], rows_v, sem).wait()

        def scale(g, _):
            ev = ex_v[pl.ds(g * 16, 16)]
            for k in range(16):
                sc = ev[k]
                r = g * 16 + k
                for j in range(W_COLS // 16):
                    sl = pl.ds(j * 16, 16)
                    rows_v[r, sl] = rows_v[r, sl] * sc
            return 0

        lax.fori_loop(0, CHUNK // 16, scale, 0)
        pltpu.sync_copy(rows_v, acc_sh.at[idx_v.at[1]], add=True)
        return 0

    lax.fori_loop(0, NCH, chunk, 0)
    plsc.subcore_barrier()
    pltpu.sync_copy(acc_sh.at[pl.ds(r0, ROWS_PT)],
                    out_hbm.at[c, pl.ds(r0, ROWS_PT)])


def _edge_phase(xpw, asrc, adst, src_p, dst_p, init):
    mesh = plsc.VectorSubcoreMesh(core_axis_name="c", subcore_axis_name="s",
                                  num_cores=NC, num_subcores=NS)
    f = pl.kernel(
        _edge_body,
        out_type=jax.ShapeDtypeStruct((2, NP, W_COLS), jnp.float32),
        mesh=mesh,
        scratch_types=[
            pltpu.VMEM((NP,), jnp.float32),
            pltpu.VMEM((NP,), jnp.float32),
            pltpu.VMEM((2, CHUNK), jnp.int32),
            pltpu.VMEM((CHUNK,), jnp.float32),
            pltpu.VMEM((CHUNK, W_COLS), jnp.float32),
            pltpu.VMEM_SHARED((NP, W_COLS), jnp.float32),
            pltpu.SemaphoreType.DMA,
        ],
        compiler_params=pltpu.CompilerParams(needs_layout_passes=False,
                                             use_tc_tiling_on_sc=False),
    )
    return f(xpw, asrc, adst, src_p, dst_p, init)


def _finish_body(part_ref, b_ref, batch_ref, lw_ref, lb_ref, out_ref):
    acc = part_ref[0] + part_ref[1]
    numer = acc[:, :H]
    denom = acc[:, H:H + 1]
    h = jnp.maximum(numer / denom + b_ref[...], 0.0)
    gids = lax.broadcasted_iota(jnp.int32, (G, NP), 0)
    oh = (gids == batch_ref[...]).astype(jnp.float32)
    sums = jnp.dot(oh, h, preferred_element_type=jnp.float32)
    cnt = jnp.sum(oh, axis=1, keepdims=True)
    pooled = sums / jnp.maximum(cnt, 1.0)
    out_ref[...] = (jnp.dot(pooled, lw_ref[...],
                            preferred_element_type=jnp.float32) + lb_ref[...])


def _finish(part, b, batch_p, linW, linb):
    return pl.pallas_call(
        _finish_body,
        out_shape=jax.ShapeDtypeStruct((G, OUT), jnp.float32),
    )(part, b.reshape(1, H), batch_p, linW, linb.reshape(1, OUT))


def kernel(x, edge_index, batch, W1, a_s1, a_d1, b1, W2, a_s2, a_d2, b2,
           linW, linb):
    del W1, a_s1, a_d1, b1
    x_p = jnp.concatenate([x, jnp.zeros((NP - N, D), jnp.float32)], axis=0)
    xpw, asrc, adst, init = _prep(x_p, W2, a_s2, a_d2)
    asrc = asrc.reshape(NP)
    adst = adst.reshape(NP)

    pad = jnp.full((NW, EPT_PAD - EPT), N, jnp.int32)
    src_p = jnp.concatenate([edge_index[0].reshape(NW, EPT), pad],
                            axis=1).reshape(-1)
    dst_p = jnp.concatenate([edge_index[1].reshape(NW, EPT), pad],
                            axis=1).reshape(-1)

    part = _edge_phase(xpw, asrc, adst, src_p, dst_p, init)

    batch_p = jnp.concatenate(
        [batch, jnp.full((NP - N,), G, jnp.int32)]).reshape(1, NP)
    return _finish(part, b2, batch_p, linW, linb)
